# baseline (speedup 1.0000x reference)
import functools

import jax
import jax.numpy as jnp
from jax import lax
from jax.experimental import pallas as pl
from jax.experimental.pallas import tpu as pltpu
from jax.experimental.pallas import tpu_sc as plsc

N_PROD = 50000
D_IN = 32
D_H = 64
N_OUT = 100
N_EDGE = 800000
CHUNK = 128
N_SUB = 16
N_PAD = 50048
ROWS_PER_TILE = N_PAD // N_SUB
EDGE_ROWS = N_EDGE // CHUNK
RING = 6
IBLK = 2
IHALVES = 6
IROT = IHALVES * IBLK


def _feat_body(ei_rt, x_dem, ei_s, x_prod, zeros32, zeros1, ones_h,
               sum_rt, cnt_rt, sum_s, cnt_s,
               acc, cnt1, rows, idx_sd, ones1, semG, semA, semI):
    cid = lax.axis_index("c")
    sid = lax.axis_index("s")
    arow0 = sid * ROWS_PER_TILE
    n_c = jnp.where(sid < 5, 392, 390)
    row0 = 390 * sid + 2 * jnp.minimum(sid, 5)
    nblk = n_c // IBLK

    pltpu.sync_copy(zeros32, acc.at[pl.ds(arow0, ROWS_PER_TILE)])
    pltpu.sync_copy(zeros1, cnt1.at[pl.ds(arow0, ROWS_PER_TILE)])
    pltpu.sync_copy(ones_h, ones1)
    plsc.subcore_barrier()

    def run(ei3, xsrc_hbm, out_hbm, cnt_hbm):
        def idx_load(blk):
            half = blk % IHALVES
            pltpu.async_copy(ei3.at[:, pl.ds(row0 + blk * IBLK, IBLK)],
                             idx_sd.at[half], semI)

        def idx_wait():
            pltpu.make_async_copy(ei3.at[:, pl.ds(row0, IBLK)],
                                  idx_sd.at[0], semI).wait()

        def add_wait():
            pltpu.make_async_copy(rows.at[0], acc.at[idx_sd.at[0, 1, 0]],
                                  semA).wait()
            pltpu.make_async_copy(ones1, cnt1.at[idx_sd.at[0, 1, 0]],
                                  semA).wait()

        for b in range(5):
            idx_load(b)
        idx_wait()
        idx_wait()
        for j in range(RING - 2):
            pltpu.async_copy(xsrc_hbm.at[idx_sd.at[j // IBLK, 0, j % IBLK]],
                             rows.at[j], semG)

        def body(c, carry):
            half = (c // IBLK) % IHALVES
            j = c % IBLK
            slot = c % RING
            pltpu.make_async_copy(xsrc_hbm.at[idx_sd.at[half, 0, j]],
                                  rows.at[slot], semG).wait()
            pltpu.async_copy(rows.at[slot], acc.at[idx_sd.at[half, 1, j]],
                             semA, add=True)
            pltpu.async_copy(ones1, cnt1.at[idx_sd.at[half, 1, j]], semA,
                             add=True)

            @pl.when(c >= 2)
            def _():
                add_wait()

            odd = c % 2 == 1

            @pl.when(odd & (c + RING - 2 < n_c))
            def _():
                idx_wait()

            @pl.when(c + RING - 2 < n_c)
            def _():
                n = c + RING - 2
                pltpu.async_copy(
                    xsrc_hbm.at[idx_sd.at[(n // IBLK) % IHALVES, 0, n % IBLK]],
                    rows.at[n % RING], semG)

            @pl.when(odd & ((c - 1) // 2 + 5 < nblk))
            def _():
                idx_load((c - 1) // 2 + 5)

            return carry

        lax.fori_loop(0, n_c, body, 0)
        add_wait()
        add_wait()
        plsc.subcore_barrier()
        pltpu.sync_copy(acc.at[pl.ds(arow0, ROWS_PER_TILE)],
                        out_hbm.at[pl.ds(arow0, ROWS_PER_TILE)])
        pltpu.sync_copy(cnt1.at[pl.ds(arow0, ROWS_PER_TILE)],
                        cnt_hbm.at[pl.ds(arow0, ROWS_PER_TILE)])

    @pl.when(cid == 0)
    def _():
        run(ei_rt, x_dem, sum_rt, cnt_rt)

    @pl.when(cid == 1)
    def _():
        run(ei_s, x_prod, sum_s, cnt_s)


@functools.partial(jax.jit, static_argnames=())
def _segment_sums(ei_rt, x_dem, ei_s, x_prod):
    zeros32 = jnp.zeros((ROWS_PER_TILE, D_IN), jnp.float32)
    zeros1 = jnp.zeros((ROWS_PER_TILE,), jnp.float32)
    ones_h = jnp.ones((CHUNK,), jnp.float32)
    mesh = plsc.VectorSubcoreMesh(core_axis_name="c", subcore_axis_name="s")
    feat = pl.kernel(
        _feat_body,
        out_type=[
            jax.ShapeDtypeStruct((N_PAD, D_IN), jnp.float32),
            jax.ShapeDtypeStruct((N_PAD,), jnp.float32),
            jax.ShapeDtypeStruct((N_PAD, D_IN), jnp.float32),
            jax.ShapeDtypeStruct((N_PAD,), jnp.float32),
        ],
        mesh=mesh,
        scratch_types=[
            pltpu.VMEM_SHARED((N_PAD, D_IN), jnp.float32),
            pltpu.VMEM_SHARED((N_PAD,), jnp.float32),
            pltpu.VMEM((RING, CHUNK, D_IN), jnp.float32),
            pltpu.VMEM((IHALVES, 2, IBLK, CHUNK), jnp.int32),
            pltpu.VMEM((CHUNK,), jnp.float32),
            pltpu.SemaphoreType.DMA,
            pltpu.SemaphoreType.DMA,
            pltpu.SemaphoreType.DMA,
        ],
        compiler_params=pltpu.CompilerParams(use_tc_tiling_on_sc=False),
    )
    return feat(ei_rt, x_dem, ei_s, x_prod, zeros32, zeros1, ones_h)


def _dense_body(s1p, cnt1, s2p, cnt2, xpp, wl1, wl2, wr, e4, bl, wlin,
                blin, out):
    scale1 = jnp.dot(1.0 / jnp.maximum(cnt1[...], 1.0), e4[...],
                     preferred_element_type=jnp.float32)
    scale2 = jnp.dot(1.0 / jnp.maximum(cnt2[...], 1.0), e4[...],
                     preferred_element_type=jnp.float32)

    def quad(x, w, d):
        return jnp.concatenate(
            [jnp.dot(x[:, d * m:d * m + d], w,
                     preferred_element_type=jnp.float32) for m in range(4)],
            axis=1)

    t = (quad(s1p[...], wl1[...], D_IN) * scale1
         + quad(s2p[...], wl2[...], D_IN) * scale2
         + quad(xpp[...], wr[...], D_IN))
    h = (t + bl[...]) * 0.5
    h = jnp.where(h >= 0, h, 0.01 * h)
    out[...] = quad(h, wlin[...], D_H) + blin[...]


def _dense(sum_rt, cnt_rt, sum_s, cnt_s, xp, wl_rt, wl_s, wr_rt, wr_s,
           bl_rt, bl_s, wlin, blin):
    rows = N_PAD // 4
    blk = rows // 23
    grid = (23,)
    spec = lambda w: pl.BlockSpec((blk, w), lambda i: (i, 0))
    full = lambda a, b: pl.BlockSpec((a, b), lambda i: (0, 0))
    e4 = jnp.repeat(jnp.eye(4, dtype=jnp.float32), D_H, axis=1)
    bl = jnp.tile(bl_rt + bl_s, 4).reshape(1, 4 * D_H)
    bl4 = jnp.tile(blin, 4).reshape(1, 4 * N_OUT)
    outp = pl.pallas_call(
        _dense_body,
        grid=grid,
        in_specs=[
            spec(4 * D_IN), spec(4), spec(4 * D_IN), spec(4), spec(4 * D_IN),
            full(D_IN, D_H), full(D_IN, D_H), full(D_IN, D_H),
            full(4, 4 * D_H),
            full(1, 4 * D_H), full(D_H, N_OUT), full(1, 4 * N_OUT),
        ],
        out_specs=spec(4 * N_OUT),
        out_shape=jax.ShapeDtypeStruct((N_PROD // 4, 4 * N_OUT), jnp.float32),
    )(sum_rt.reshape(rows, 4 * D_IN), cnt_rt.reshape(rows, 4),
      sum_s.reshape(rows, 4 * D_IN), cnt_s.reshape(rows, 4),
      xp.reshape(N_PROD // 4, 4 * D_IN), wl_rt, wl_s, wr_rt + wr_s, e4, bl,
      wlin, bl4)
    return outp.reshape(N_PROD, N_OUT)


def kernel(x_product, x_demographic, x_platform, edge_index_targets,
           edge_index_rev_targets, edge_index_uses, edge_index_rev_uses,
           edge_index_self,
           Wl_t, bl_t, Wr_t,
           Wl_rt, bl_rt, Wr_rt,
           Wl_u, bl_u, Wr_u,
           Wl_ru, bl_ru, Wr_ru,
           Wl_s, bl_s, Wr_s,
           W_lin, b_lin):
    ei_rt = edge_index_rev_targets.reshape(2, EDGE_ROWS, CHUNK)
    ei_s = edge_index_self.reshape(2, EDGE_ROWS, CHUNK)
    sum_rt, cnt_rt, sum_s, cnt_s = _segment_sums(
        ei_rt, x_demographic, ei_s, x_product)
    return _dense(sum_rt, cnt_rt, sum_s, cnt_s, x_product,
                  Wl_rt, Wl_s, Wr_rt, Wr_s, bl_rt, bl_s, W_lin, b_lin)


# speedup vs baseline: 1.0401x; 1.0401x over previous
import functools

import jax
import jax.numpy as jnp
from jax import lax
from jax.experimental import pallas as pl
from jax.experimental.pallas import tpu as pltpu
from jax.experimental.pallas import tpu_sc as plsc

N_PROD = 50000
D_IN = 32
D_H = 64
N_OUT = 100
N_EDGE = 800000
CHUNK = 128
N_SUB = 16
N_PAD = 50048
ROWS_PER_TILE = N_PAD // N_SUB
EDGE_ROWS = N_EDGE // CHUNK
RING = 6
IBLK = 2
IHALVES = 6
IROT = IHALVES * IBLK


def _feat_body(ei_rt, x_dem, ei_s, x_prod, zeros32, zeros1, ones_h,
               sum_rt, cnt_rt, sum_s, cnt_s,
               acc, cnt1, rows, idx_sd, ones1, semG, semA, semI):
    cid = lax.axis_index("c")
    sid = lax.axis_index("s")
    arow0 = sid * ROWS_PER_TILE
    n_c = jnp.where(sid < 5, 392, 390)
    row0 = 390 * sid + 2 * jnp.minimum(sid, 5)
    nblk = n_c // IBLK

    pltpu.sync_copy(zeros32, acc.at[pl.ds(arow0, ROWS_PER_TILE)])
    pltpu.sync_copy(zeros1, cnt1.at[pl.ds(arow0, ROWS_PER_TILE)])
    pltpu.sync_copy(ones_h, ones1)
    plsc.subcore_barrier()

    def run(ei3, xsrc_hbm, out_hbm, cnt_hbm):
        def idx_load(blk):
            half = blk % IHALVES
            pltpu.async_copy(ei3.at[:, pl.ds(row0 + blk * IBLK, IBLK)],
                             idx_sd.at[half], semI)

        def idx_wait():
            pltpu.make_async_copy(ei3.at[:, pl.ds(row0, IBLK)],
                                  idx_sd.at[0], semI).wait()

        def add_wait():
            pltpu.make_async_copy(rows.at[0], acc.at[idx_sd.at[0, 1, 0]],
                                  semA).wait()
            pltpu.make_async_copy(ones1, cnt1.at[idx_sd.at[0, 1, 0]],
                                  semA).wait()

        for b in range(5):
            idx_load(b)
        idx_wait()
        idx_wait()
        for j in range(RING - 2):
            pltpu.async_copy(xsrc_hbm.at[idx_sd.at[j // IBLK, 0, j % IBLK]],
                             rows.at[j], semG)

        def body(c, carry):
            half = (c // IBLK) % IHALVES
            j = c % IBLK
            slot = c % RING
            pltpu.make_async_copy(xsrc_hbm.at[idx_sd.at[half, 0, j]],
                                  rows.at[slot], semG).wait()
            pltpu.async_copy(rows.at[slot], acc.at[idx_sd.at[half, 1, j]],
                             semA, add=True)
            pltpu.async_copy(ones1, cnt1.at[idx_sd.at[half, 1, j]], semA,
                             add=True)

            @pl.when(c >= 2)
            def _():
                add_wait()

            odd = c % 2 == 1

            @pl.when(odd & (c + RING - 2 < n_c))
            def _():
                idx_wait()

            @pl.when(c + RING - 2 < n_c)
            def _():
                n = c + RING - 2
                pltpu.async_copy(
                    xsrc_hbm.at[idx_sd.at[(n // IBLK) % IHALVES, 0, n % IBLK]],
                    rows.at[n % RING], semG)

            @pl.when(odd & ((c - 1) // 2 + 5 < nblk))
            def _():
                idx_load((c - 1) // 2 + 5)

            return carry

        lax.fori_loop(0, n_c, body, 0)
        add_wait()
        add_wait()
        plsc.subcore_barrier()
        pltpu.sync_copy(acc.at[pl.ds(arow0, ROWS_PER_TILE)],
                        out_hbm.at[pl.ds(arow0, ROWS_PER_TILE)])
        pltpu.sync_copy(cnt1.at[pl.ds(arow0, ROWS_PER_TILE)],
                        cnt_hbm.at[pl.ds(arow0, ROWS_PER_TILE)])

    @pl.when(cid == 0)
    def _():
        run(ei_rt, x_dem, sum_rt, cnt_rt)

    @pl.when(cid == 1)
    def _():
        run(ei_s, x_prod, sum_s, cnt_s)


@functools.partial(jax.jit, static_argnames=())
def _segment_sums(ei_rt, x_dem, ei_s, x_prod):
    zeros32 = jnp.zeros((ROWS_PER_TILE, D_IN), jnp.float32)
    zeros1 = jnp.zeros((ROWS_PER_TILE,), jnp.float32)
    ones_h = jnp.ones((CHUNK,), jnp.float32)
    mesh = plsc.VectorSubcoreMesh(core_axis_name="c", subcore_axis_name="s")
    feat = pl.kernel(
        _feat_body,
        out_type=[
            jax.ShapeDtypeStruct((N_PAD, D_IN), jnp.float32),
            jax.ShapeDtypeStruct((N_PAD,), jnp.float32),
            jax.ShapeDtypeStruct((N_PAD, D_IN), jnp.float32),
            jax.ShapeDtypeStruct((N_PAD,), jnp.float32),
        ],
        mesh=mesh,
        scratch_types=[
            pltpu.VMEM_SHARED((N_PAD, D_IN), jnp.float32),
            pltpu.VMEM_SHARED((N_PAD,), jnp.float32),
            pltpu.VMEM((RING, CHUNK, D_IN), jnp.float32),
            pltpu.VMEM((IHALVES, 2, IBLK, CHUNK), jnp.int32),
            pltpu.VMEM((CHUNK,), jnp.float32),
            pltpu.SemaphoreType.DMA,
            pltpu.SemaphoreType.DMA,
            pltpu.SemaphoreType.DMA,
        ],
        compiler_params=pltpu.CompilerParams(use_tc_tiling_on_sc=False),
    )
    return feat(ei_rt, x_dem, ei_s, x_prod, zeros32, zeros1, ones_h)


def _dense_body(s1p, rec1, s2p, rec2, xpp, w1, w2, wr, e4, bl, wlin, blin,
                out):
    scale1 = jnp.dot(1.0 / jnp.maximum(rec1[...], 1.0), e4[...],
                     preferred_element_type=jnp.float32)
    scale2 = jnp.dot(1.0 / jnp.maximum(rec2[...], 1.0), e4[...],
                     preferred_element_type=jnp.float32)
    t = (jnp.dot(s1p[...], w1[...], preferred_element_type=jnp.float32)
         * scale1
         + jnp.dot(s2p[...], w2[...], preferred_element_type=jnp.float32)
         * scale2
         + jnp.dot(xpp[...], wr[...], preferred_element_type=jnp.float32))
    h = (t + bl[...]) * 0.5
    h = jnp.where(h >= 0, h, 0.01 * h)
    out[...] = jnp.dot(h, wlin[...],
                       preferred_element_type=jnp.float32) + blin[...]


def _kron4(w):
    z = jnp.zeros_like(w)
    r1 = jnp.concatenate([w, z, z, z], axis=1)
    r2 = jnp.concatenate([z, w, z, z], axis=1)
    r3 = jnp.concatenate([z, z, w, z], axis=1)
    r4 = jnp.concatenate([z, z, z, w], axis=1)
    return jnp.concatenate([r1, r2, r3, r4], axis=0)


def _dense(sum_rt, cnt_rt, sum_s, cnt_s, xp, wl_rt, wl_s, wr_rt, wr_s,
           bl_rt, bl_s, wlin, blin):
    rows = N_PAD // 4
    blk = rows // 23
    grid = (23,)
    spec = lambda w: pl.BlockSpec((blk, w), lambda i: (i, 0))
    full = lambda a, b: pl.BlockSpec((a, b), lambda i: (0, 0))
    w1 = _kron4(wl_rt)
    w2 = _kron4(wl_s)
    wr = _kron4(wr_rt + wr_s)
    wlin4 = _kron4(wlin)
    e4 = jnp.repeat(jnp.eye(4, dtype=jnp.float32), D_H, axis=1)
    bl = jnp.tile(bl_rt + bl_s, 4).reshape(1, 4 * D_H)
    bl4 = jnp.tile(blin, 4).reshape(1, 4 * N_OUT)
    outp = pl.pallas_call(
        _dense_body,
        grid=grid,
        in_specs=[
            spec(4 * D_IN), spec(4), spec(4 * D_IN), spec(4), spec(4 * D_IN),
            full(4 * D_IN, 4 * D_H), full(4 * D_IN, 4 * D_H),
            full(4 * D_IN, 4 * D_H), full(4, 4 * D_H),
            full(1, 4 * D_H), full(4 * D_H, 4 * N_OUT), full(1, 4 * N_OUT),
        ],
        out_specs=spec(4 * N_OUT),
        out_shape=jax.ShapeDtypeStruct((N_PROD // 4, 4 * N_OUT), jnp.float32),
    )(sum_rt.reshape(rows, 4 * D_IN), cnt_rt.reshape(rows, 4),
      sum_s.reshape(rows, 4 * D_IN), cnt_s.reshape(rows, 4),
      xp.reshape(N_PROD // 4, 4 * D_IN), w1, w2, wr, e4, bl, wlin4, bl4)
    return outp.reshape(N_PROD, N_OUT)


def kernel(x_product, x_demographic, x_platform, edge_index_targets,
           edge_index_rev_targets, edge_index_uses, edge_index_rev_uses,
           edge_index_self,
           Wl_t, bl_t, Wr_t,
           Wl_rt, bl_rt, Wr_rt,
           Wl_u, bl_u, Wr_u,
           Wl_ru, bl_ru, Wr_ru,
           Wl_s, bl_s, Wr_s,
           W_lin, b_lin):
    ei_rt = edge_index_rev_targets.reshape(2, EDGE_ROWS, CHUNK)
    ei_s = edge_index_self.reshape(2, EDGE_ROWS, CHUNK)
    sum_rt, cnt_rt, sum_s, cnt_s = _segment_sums(
        ei_rt, x_demographic, ei_s, x_product)
    return _dense(sum_rt, cnt_rt, sum_s, cnt_s, x_product,
                  Wl_rt, Wl_s, Wr_rt, Wr_s, bl_rt, bl_s, W_lin, b_lin)


# speedup vs baseline: 1.0421x; 1.0019x over previous
import functools

import jax
import jax.numpy as jnp
from jax import lax
from jax.experimental import pallas as pl
from jax.experimental.pallas import tpu as pltpu
from jax.experimental.pallas import tpu_sc as plsc

N_PROD = 50000
D_IN = 32
D_H = 64
N_OUT = 100
N_EDGE = 800000
CHUNK = 128
N_SUB = 16
N_PAD = 50048
ROWS_PER_TILE = N_PAD // N_SUB
EDGE_ROWS = N_EDGE // CHUNK
RING = 6
IBLK = 2
IHALVES = 6
IROT = IHALVES * IBLK


def _feat_body(ei_rt, x_dem, ei_s, x_prod, zeros32, zeros1, ones_h,
               sum_rt, cnt_rt, sum_s, cnt_s,
               acc, cnt1, rows, idx_sd, ones1, semG, semA, semI):
    cid = lax.axis_index("c")
    sid = lax.axis_index("s")
    arow0 = sid * ROWS_PER_TILE
    n_c = jnp.where(sid < 5, 392, 390)
    row0 = 390 * sid + 2 * jnp.minimum(sid, 5)
    nblk = n_c // IBLK

    pltpu.sync_copy(zeros32, acc.at[pl.ds(arow0, ROWS_PER_TILE)])
    pltpu.sync_copy(zeros1, cnt1.at[pl.ds(arow0, ROWS_PER_TILE)])
    pltpu.sync_copy(ones_h, ones1)
    plsc.subcore_barrier()

    def run(ei3, xsrc_hbm, out_hbm, cnt_hbm):
        def idx_load(blk):
            half = blk % IHALVES
            pltpu.async_copy(ei3.at[:, pl.ds(row0 + blk * IBLK, IBLK)],
                             idx_sd.at[half], semI)

        def idx_wait():
            pltpu.make_async_copy(ei3.at[:, pl.ds(row0, IBLK)],
                                  idx_sd.at[0], semI).wait()

        def add_wait():
            pltpu.make_async_copy(rows.at[0], acc.at[idx_sd.at[0, 1, 0]],
                                  semA).wait()
            pltpu.make_async_copy(ones1, cnt1.at[idx_sd.at[0, 1, 0]],
                                  semA).wait()

        for b in range(5):
            idx_load(b)
        idx_wait()
        idx_wait()
        for j in range(RING - 2):
            pltpu.async_copy(xsrc_hbm.at[idx_sd.at[j // IBLK, 0, j % IBLK]],
                             rows.at[j], semG)

        def body(c, carry):
            half = (c // IBLK) % IHALVES
            j = c % IBLK
            slot = c % RING
            pltpu.make_async_copy(xsrc_hbm.at[idx_sd.at[half, 0, j]],
                                  rows.at[slot], semG).wait()
            pltpu.async_copy(rows.at[slot], acc.at[idx_sd.at[half, 1, j]],
                             semA, add=True)
            pltpu.async_copy(ones1, cnt1.at[idx_sd.at[half, 1, j]], semA,
                             add=True)

            @pl.when(c >= 2)
            def _():
                add_wait()

            odd = c % 2 == 1

            @pl.when(odd & (c + RING - 2 < n_c))
            def _():
                idx_wait()

            @pl.when(c + RING - 2 < n_c)
            def _():
                n = c + RING - 2
                pltpu.async_copy(
                    xsrc_hbm.at[idx_sd.at[(n // IBLK) % IHALVES, 0, n % IBLK]],
                    rows.at[n % RING], semG)

            @pl.when(odd & ((c - 1) // 2 + 5 < nblk))
            def _():
                idx_load((c - 1) // 2 + 5)

            return carry

        lax.fori_loop(0, n_c, body, 0)
        add_wait()
        add_wait()
        plsc.subcore_barrier()
        pltpu.sync_copy(acc.at[pl.ds(arow0, ROWS_PER_TILE)],
                        out_hbm.at[pl.ds(arow0, ROWS_PER_TILE)])
        pltpu.sync_copy(cnt1.at[pl.ds(arow0, ROWS_PER_TILE)],
                        cnt_hbm.at[pl.ds(arow0, ROWS_PER_TILE)])

    @pl.when(cid == 0)
    def _():
        run(ei_rt, x_dem, sum_rt, cnt_rt)

    @pl.when(cid == 1)
    def _():
        run(ei_s, x_prod, sum_s, cnt_s)


@functools.partial(jax.jit, static_argnames=())
def _segment_sums(ei_rt, x_dem, ei_s, x_prod):
    zeros32 = jnp.zeros((ROWS_PER_TILE, D_IN), jnp.float32)
    zeros1 = jnp.zeros((ROWS_PER_TILE,), jnp.float32)
    ones_h = jnp.ones((CHUNK,), jnp.float32)
    mesh = plsc.VectorSubcoreMesh(core_axis_name="c", subcore_axis_name="s")
    feat = pl.kernel(
        _feat_body,
        out_type=[
            jax.ShapeDtypeStruct((N_PAD, D_IN), jnp.float32),
            jax.ShapeDtypeStruct((N_PAD,), jnp.float32),
            jax.ShapeDtypeStruct((N_PAD, D_IN), jnp.float32),
            jax.ShapeDtypeStruct((N_PAD,), jnp.float32),
        ],
        mesh=mesh,
        scratch_types=[
            pltpu.VMEM_SHARED((N_PAD, D_IN), jnp.float32),
            pltpu.VMEM_SHARED((N_PAD,), jnp.float32),
            pltpu.VMEM((RING, CHUNK, D_IN), jnp.float32),
            pltpu.VMEM((IHALVES, 2, IBLK, CHUNK), jnp.int32),
            pltpu.VMEM((CHUNK,), jnp.float32),
            pltpu.SemaphoreType.DMA,
            pltpu.SemaphoreType.DMA,
            pltpu.SemaphoreType.DMA,
        ],
        compiler_params=pltpu.CompilerParams(use_tc_tiling_on_sc=False),
    )
    return feat(ei_rt, x_dem, ei_s, x_prod, zeros32, zeros1, ones_h)


def _dense_body(s1p, cnt1, s2p, cnt2, xpp, w1, w2, wr, e4, bl, wlin, blin,
                out):
    scale1 = jnp.dot(1.0 / jnp.maximum(cnt1[...], 1.0), e4[...],
                     preferred_element_type=jnp.float32)
    scale2 = jnp.dot(1.0 / jnp.maximum(cnt2[...], 1.0), e4[...],
                     preferred_element_type=jnp.float32)
    t = (jnp.dot(s1p[...], w1[...], preferred_element_type=jnp.float32)
         * scale1
         + jnp.dot(s2p[...], w2[...], preferred_element_type=jnp.float32)
         * scale2
         + jnp.dot(xpp[...], wr[...], preferred_element_type=jnp.float32))
    h = (t + bl[...]) * 0.5
    h = jnp.where(h >= 0, h, 0.01 * h)
    out[...] = jnp.dot(h, wlin[...],
                       preferred_element_type=jnp.float32) + blin[...]


def _kron4(w):
    z = jnp.zeros_like(w)
    r1 = jnp.concatenate([w, z, z, z], axis=1)
    r2 = jnp.concatenate([z, w, z, z], axis=1)
    r3 = jnp.concatenate([z, z, w, z], axis=1)
    r4 = jnp.concatenate([z, z, z, w], axis=1)
    return jnp.concatenate([r1, r2, r3, r4], axis=0)


def _dense(sum_rt, cnt_rt, sum_s, cnt_s, xp, wl_rt, wl_s, wr_rt, wr_s,
           bl_rt, bl_s, wlin, blin):
    rows = N_PAD // 4
    blk = rows // 23
    grid = (23,)
    spec = lambda w: pl.BlockSpec((blk, w), lambda i: (i, 0))
    full = lambda a, b: pl.BlockSpec((a, b), lambda i: (0, 0))
    w1 = _kron4(wl_rt)
    w2 = _kron4(wl_s)
    wr = _kron4(wr_rt + wr_s)
    wlin4 = _kron4(wlin)
    e4 = jnp.repeat(jnp.eye(4, dtype=jnp.float32), D_H, axis=1)
    bl = jnp.tile(bl_rt + bl_s, 4).reshape(1, 4 * D_H)
    bl4 = jnp.tile(blin, 4).reshape(1, 4 * N_OUT)
    outp = pl.pallas_call(
        _dense_body,
        grid=grid,
        in_specs=[
            spec(4 * D_IN), spec(4), spec(4 * D_IN), spec(4), spec(4 * D_IN),
            full(4 * D_IN, 4 * D_H), full(4 * D_IN, 4 * D_H),
            full(4 * D_IN, 4 * D_H), full(4, 4 * D_H),
            full(1, 4 * D_H), full(4 * D_H, 4 * N_OUT), full(1, 4 * N_OUT),
        ],
        out_specs=spec(4 * N_OUT),
        out_shape=jax.ShapeDtypeStruct((N_PROD // 4, 4 * N_OUT), jnp.float32),
    )(sum_rt.reshape(rows, 4 * D_IN), cnt_rt.reshape(rows, 4),
      sum_s.reshape(rows, 4 * D_IN), cnt_s.reshape(rows, 4),
      xp.reshape(N_PROD // 4, 4 * D_IN), w1, w2, wr, e4, bl, wlin4, bl4)
    return outp.reshape(N_PROD, N_OUT)


def kernel(x_product, x_demographic, x_platform, edge_index_targets,
           edge_index_rev_targets, edge_index_uses, edge_index_rev_uses,
           edge_index_self,
           Wl_t, bl_t, Wr_t,
           Wl_rt, bl_rt, Wr_rt,
           Wl_u, bl_u, Wr_u,
           Wl_ru, bl_ru, Wr_ru,
           Wl_s, bl_s, Wr_s,
           W_lin, b_lin):
    ei_rt = edge_index_rev_targets.reshape(2, EDGE_ROWS, CHUNK)
    ei_s = edge_index_self.reshape(2, EDGE_ROWS, CHUNK)
    sum_rt, cnt_rt, sum_s, cnt_s = _segment_sums(
        ei_rt, x_demographic, ei_s, x_product)
    return _dense(sum_rt, cnt_rt, sum_s, cnt_s, x_product,
                  Wl_rt, Wl_s, Wr_rt, Wr_s, bl_rt, bl_s, W_lin, b_lin)
